# baseline (speedup 1.0000x reference)
import functools

import jax
import jax.numpy as jnp
from jax.experimental import pallas as pl
from jax.experimental.pallas import tpu as pltpu


def _gate_kernel(x_ref, w_ref, idx_ref, wgt_ref):
    logits = jnp.dot(x_ref[:, :], w_ref[:, :], preferred_element_type=jnp.float32)
    e = logits.shape[-1]
    iota = jax.lax.broadcasted_iota(jnp.int32, logits.shape, 1)
    m1 = jnp.max(logits, axis=1, keepdims=True)
    i1 = jnp.min(jnp.where(logits == m1, iota, e), axis=1, keepdims=True)
    masked = jnp.where(iota == i1, -jnp.inf, logits)
    m2 = jnp.max(masked, axis=1, keepdims=True)
    i2 = jnp.min(jnp.where(masked == m2, iota, e), axis=1, keepdims=True)
    t = jnp.exp(m2 - m1)
    w1 = 1.0 / (1.0 + t)
    idx_ref[:, :] = jnp.concatenate([i1, i2], axis=1)
    wgt_ref[:, :] = jnp.concatenate([w1, 1.0 - w1], axis=1)


@functools.partial(jax.jit, static_argnames=("tile",))
def _gate(xf, wt, tile):
    n, d = xf.shape
    e = wt.shape[1]
    idx, wgt = pl.pallas_call(
        _gate_kernel,
        grid=(n // tile,),
        in_specs=[
            pl.BlockSpec((tile, d), lambda i: (i, 0)),
            pl.BlockSpec((d, e), lambda i: (0, 0)),
        ],
        out_specs=[
            pl.BlockSpec((tile, 2), lambda i: (i, 0)),
            pl.BlockSpec((tile, 2), lambda i: (i, 0)),
        ],
        out_shape=[
            jax.ShapeDtypeStruct((n, 2), jnp.int32),
            jax.ShapeDtypeStruct((n, 2), jnp.float32),
        ],
        compiler_params=pltpu.CompilerParams(
            dimension_semantics=("parallel",),
        ),
    )(xf, wt)
    return idx, wgt


def kernel(x, w_gate):
    b, t, d = x.shape
    xf = x.reshape(b * t, d)
    wt = w_gate.T
    idx, wgt = _gate(xf, wt, tile=512)
    return idx.reshape(b, t, 2), wgt.reshape(b, t, 2)


# speedup vs baseline: 1.2846x; 1.2846x over previous
import functools

import jax
import jax.numpy as jnp
from jax.experimental import pallas as pl
from jax.experimental.pallas import tpu as pltpu


def _gate_kernel(x_ref, w_ref, idx_ref, wgt_ref):
    logits = jnp.dot(x_ref[:, :], w_ref[:, :], preferred_element_type=jnp.float32)
    e = logits.shape[-1]
    iota = jax.lax.broadcasted_iota(jnp.int32, logits.shape, 1)
    m1 = jnp.max(logits, axis=1, keepdims=True)
    i1 = jnp.min(jnp.where(logits == m1, iota, e), axis=1, keepdims=True)
    masked = jnp.where(iota == i1, -jnp.inf, logits)
    m2 = jnp.max(masked, axis=1, keepdims=True)
    i2 = jnp.min(jnp.where(masked == m2, iota, e), axis=1, keepdims=True)
    t = jnp.exp(m2 - m1)
    w1 = 1.0 / (1.0 + t)
    idx_ref[:, :] = jnp.concatenate([i1, i2], axis=1)
    wgt_ref[:, :] = jnp.concatenate([w1, 1.0 - w1], axis=1)


@functools.partial(jax.jit, static_argnames=("tile",))
def _gate(xf, wt, tile):
    n, d = xf.shape
    e = wt.shape[1]
    idx, wgt = pl.pallas_call(
        _gate_kernel,
        grid=(n // tile,),
        in_specs=[
            pl.BlockSpec((tile, d), lambda i: (i, 0)),
            pl.BlockSpec((d, e), lambda i: (0, 0)),
        ],
        out_specs=[
            pl.BlockSpec((tile, 2), lambda i: (i, 0)),
            pl.BlockSpec((tile, 2), lambda i: (i, 0)),
        ],
        out_shape=[
            jax.ShapeDtypeStruct((n, 2), jnp.int32),
            jax.ShapeDtypeStruct((n, 2), jnp.float32),
        ],
        compiler_params=pltpu.CompilerParams(
            dimension_semantics=("parallel",),
        ),
    )(xf, wt)
    return idx, wgt


def kernel(x, w_gate):
    b, t, d = x.shape
    xf = x.reshape(b * t, d)
    wt = w_gate.T
    idx, wgt = _gate(xf, wt, tile=1024)
    return idx.reshape(b, t, 2), wgt.reshape(b, t, 2)


# speedup vs baseline: 1.4254x; 1.1096x over previous
import functools

import jax
import jax.numpy as jnp
from jax.experimental import pallas as pl
from jax.experimental.pallas import tpu as pltpu


def _gate_kernel(x_ref, w_ref, idx_ref, wgt_ref):
    logits = jnp.dot(x_ref[:, :], w_ref[:, :], preferred_element_type=jnp.float32)
    e = logits.shape[-1]
    iota = jax.lax.broadcasted_iota(jnp.int32, logits.shape, 1)
    m1 = jnp.max(logits, axis=1, keepdims=True)
    i1 = jnp.min(jnp.where(logits == m1, iota, e), axis=1, keepdims=True)
    masked = jnp.where(iota == i1, -jnp.inf, logits)
    m2 = jnp.max(masked, axis=1, keepdims=True)
    i2 = jnp.min(jnp.where(masked == m2, iota, e), axis=1, keepdims=True)
    t = jnp.exp(m2 - m1)
    w1 = 1.0 / (1.0 + t)
    idx_ref[:, :] = jnp.concatenate([i1, i2], axis=1)
    wgt_ref[:, :] = jnp.concatenate([w1, 1.0 - w1], axis=1)


@functools.partial(jax.jit, static_argnames=("tile",))
def _gate(xf, wt, tile):
    n, d = xf.shape
    e = wt.shape[1]
    idx, wgt = pl.pallas_call(
        _gate_kernel,
        grid=(n // tile,),
        in_specs=[
            pl.BlockSpec((tile, d), lambda i: (i, 0)),
            pl.BlockSpec((d, e), lambda i: (0, 0)),
        ],
        out_specs=[
            pl.BlockSpec((tile, 2), lambda i: (i, 0)),
            pl.BlockSpec((tile, 2), lambda i: (i, 0)),
        ],
        out_shape=[
            jax.ShapeDtypeStruct((n, 2), jnp.int32),
            jax.ShapeDtypeStruct((n, 2), jnp.float32),
        ],
        compiler_params=pltpu.CompilerParams(
            dimension_semantics=("parallel",),
        ),
    )(xf, wt)
    return idx, wgt


def kernel(x, w_gate):
    b, t, d = x.shape
    xf = x.reshape(b * t, d)
    wt = w_gate.T
    idx, wgt = _gate(xf, wt, tile=2048)
    return idx.reshape(b, t, 2), wgt.reshape(b, t, 2)


# speedup vs baseline: 1.4941x; 1.0482x over previous
import functools

import jax
import jax.numpy as jnp
from jax.experimental import pallas as pl
from jax.experimental.pallas import tpu as pltpu


def _gate_kernel(x_ref, w_ref, idx_ref, wgt_ref):
    logits = jnp.dot(x_ref[:, :], w_ref[:, :], preferred_element_type=jnp.float32)
    e = logits.shape[-1]
    iota = jax.lax.broadcasted_iota(jnp.int32, logits.shape, 1)
    m1 = jnp.max(logits, axis=1, keepdims=True)
    i1 = jnp.min(jnp.where(logits == m1, iota, e), axis=1, keepdims=True)
    masked = jnp.where(iota == i1, -jnp.inf, logits)
    m2 = jnp.max(masked, axis=1, keepdims=True)
    i2 = jnp.min(jnp.where(masked == m2, iota, e), axis=1, keepdims=True)
    t = jnp.exp(m2 - m1)
    w1 = 1.0 / (1.0 + t)
    idx_ref[:, :] = jnp.concatenate([i1, i2], axis=1)
    wgt_ref[:, :] = jnp.concatenate([w1, 1.0 - w1], axis=1)


@functools.partial(jax.jit, static_argnames=("tile",))
def _gate(xf, wt, tile):
    n, d = xf.shape
    e = wt.shape[1]
    idx, wgt = pl.pallas_call(
        _gate_kernel,
        grid=(n // tile,),
        in_specs=[
            pl.BlockSpec((tile, d), lambda i: (i, 0)),
            pl.BlockSpec((d, e), lambda i: (0, 0)),
        ],
        out_specs=[
            pl.BlockSpec((tile, 2), lambda i: (i, 0)),
            pl.BlockSpec((tile, 2), lambda i: (i, 0)),
        ],
        out_shape=[
            jax.ShapeDtypeStruct((n, 2), jnp.int32),
            jax.ShapeDtypeStruct((n, 2), jnp.float32),
        ],
        compiler_params=pltpu.CompilerParams(
            dimension_semantics=("parallel",),
        ),
    )(xf, wt)
    return idx, wgt


def kernel(x, w_gate):
    b, t, d = x.shape
    xf = x.reshape(b * t, d)
    wt = w_gate.T
    idx, wgt = _gate(xf, wt, tile=4096)
    return idx.reshape(b, t, 2), wgt.reshape(b, t, 2)
